# baseline (speedup 1.0000x reference)
import jax
import jax.numpy as jnp
from jax import lax
from jax.experimental import pallas as pl
from jax.experimental.pallas import tpu as pltpu
from jax.experimental.pallas import tpu_sc as plsc

N = 10000
NP = 10240
E = 320000
BATCH = 128
TB = E // BATCH
CAP = 80
DBPT = 157
STRIPE = NP // 16
S = 1000
SP = 1024
SPT = SP // 16

_MESH = plsc.VectorSubcoreMesh(
    core_axis_name="c", subcore_axis_name="s", num_cores=2, num_subcores=16)
_SC_PARAMS = pltpu.CompilerParams(use_tc_tiling_on_sc=False)


def _deg_body(ei3, zeros1, out, idx_v, ones_v, acc, sem):
    cid = lax.axis_index("c")
    sid = lax.axis_index("s")
    for i in range(8):
        ones_v[pl.ds(i * 16, 16)] = jnp.full((16,), 1.0, jnp.float32)
    pltpu.sync_copy(zeros1, acc.at[pl.ds(sid * STRIPE, STRIPE)])
    plane = jnp.where(cid == 0, 1, 0)
    base = 156 * sid + jnp.minimum(sid, 4)
    nb = jnp.where(sid < 4, 157, 156)

    @pl.when(sid < 4)
    def _():
        pltpu.sync_copy(ei3.at[plane, pl.ds(base, 157)], idx_v)

    @pl.when(sid >= 4)
    def _():
        pltpu.sync_copy(ei3.at[plane, pl.ds(base, 156)], idx_v.at[pl.ds(0, 156)])

    plsc.subcore_barrier()

    def body(j, carry):
        @pl.when(j < nb)
        def _():
            pltpu.async_copy(ones_v, acc.at[idx_v.at[j]], sem, add=True)
        return carry

    lax.fori_loop(0, DBPT, body, 0)

    def drain(j, carry):
        @pl.when(j < nb)
        def _():
            pltpu.make_async_copy(ones_v, acc.at[idx_v.at[j]], sem).wait()
        return carry

    lax.fori_loop(0, DBPT, drain, 0)
    plsc.subcore_barrier()
    pltpu.sync_copy(acc.at[pl.ds(sid * STRIPE, STRIPE)],
                    out.at[cid, pl.ds(sid * STRIPE, STRIPE)])


def _sweep_prologue(wid, ei3, idxs, idxd):
    base = 80 * jnp.minimum(wid, 17) + 76 * jnp.maximum(wid - 17, 0)
    nb = jnp.where(wid < 17, CAP, CAP - 4)

    @pl.when(wid < 17)
    def _():
        pltpu.sync_copy(ei3.at[0, pl.ds(base, CAP)], idxs)
        pltpu.sync_copy(ei3.at[1, pl.ds(base, CAP)], idxd)

    @pl.when(wid >= 17)
    def _():
        pltpu.sync_copy(ei3.at[0, pl.ds(base, CAP - 4)], idxs.at[pl.ds(0, CAP - 4)])
        pltpu.sync_copy(ei3.at[1, pl.ds(base, CAP - 4)], idxd.at[pl.ds(0, CAP - 4)])

    return nb


def _edge_sweep(feat, idxs, idxd, rows, acc, sg, ss, nb):
    nd = len(rows)
    la = nd // 2
    for b in range(la):
        pltpu.async_copy(feat.at[idxs.at[b]], rows[b], sg[b])

    def body(qq, carry):
        j0 = nd * qq
        for b in range(nd):
            j = j0 + b

            @pl.when(j < nb)
            def _(b=b, j=j):
                pltpu.make_async_copy(feat.at[idxs.at[j]], rows[b], sg[b]).wait()
                pltpu.async_copy(rows[b], acc.at[idxd.at[j]], ss[b], add=True)
                b8 = (b + la) % nd

                @pl.when(j + la < nb)
                def _(b=b, j=j, b8=b8):
                    @pl.when(j >= la)
                    def _():
                        pltpu.make_async_copy(
                            rows[b8], acc.at[idxd.at[j - la]], ss[b8]).wait()

                    pltpu.async_copy(feat.at[idxs.at[j + la]], rows[b8], sg[b8])

        return carry

    lax.fori_loop(0, (CAP + nd - 1) // nd, body, 0)
    for b in range(nd):
        pltpu.make_async_copy(rows[b], acc.at[idxd.at[0]], ss[b]).wait()


def _prop_body(feat, ei3, zerosf, out, *scr):
    idxs, idxd = scr[0], scr[1]
    rows, acc = list(scr[2:10]), scr[10]
    sg, ss = list(scr[11:19]), list(scr[19:27])
    cid = lax.axis_index("c")
    sid = lax.axis_index("s")
    wid = cid * 16 + sid
    pltpu.sync_copy(zerosf, acc.at[pl.ds(sid * STRIPE, STRIPE)])
    nb = _sweep_prologue(wid, ei3, idxs, idxd)
    plsc.subcore_barrier()
    _edge_sweep(feat, idxs, idxd, rows, acc, sg, ss, nb)
    plsc.subcore_barrier()
    pltpu.sync_copy(acc.at[pl.ds(sid * STRIPE, STRIPE)],
                    out.at[cid, pl.ds(sid * STRIPE, STRIPE)])


def _prop_gather_body(feat, ei3, zerosf, bcol, sampw, gpart, bg, *scr):
    idxs, idxd = scr[0], scr[1]
    rows, acc = list(scr[2:10]), scr[10]
    sidx, srows, brows = scr[11], scr[12], scr[13]
    sg, ss = list(scr[14:22]), list(scr[22:30])
    cid = lax.axis_index("c")
    sid = lax.axis_index("s")
    wid = cid * 16 + sid
    pltpu.sync_copy(zerosf, acc.at[pl.ds(sid * STRIPE, STRIPE)])
    nb = _sweep_prologue(wid, ei3, idxs, idxd)
    pltpu.sync_copy(sampw.at[sid], sidx)
    plsc.subcore_barrier()
    _edge_sweep(feat, idxs, idxd, rows, acc, sg, ss, nb)
    plsc.subcore_barrier()
    pltpu.async_copy(acc.at[sidx], srows, sg[0]).wait()
    pltpu.sync_copy(srows, gpart.at[cid, pl.ds(sid * SPT, SPT)])

    @pl.when(cid == 0)
    def _():
        pltpu.async_copy(bcol.at[sidx], brows, sg[1]).wait()
        pltpu.sync_copy(brows, bg.at[pl.ds(sid * SPT, SPT)])


def _mm1a_body(x_ref, w_ref, mm_ref):
    mm_ref[...] = jnp.dot(x_ref[...], w_ref[...],
                          preferred_element_type=jnp.float32)


def _mm1b_body(mm_ref, deg_ref, hp1_ref, a_ref, b_ref, bcol_ref):
    deg = deg_ref[...]
    a = lax.rsqrt(jnp.maximum(deg[1], 1.0)).reshape(-1, 1)
    b = lax.rsqrt(jnp.maximum(deg[0], 1.0)).reshape(-1, 1)
    hp1_ref[...] = mm_ref[...] * a
    a_ref[...] = a
    b_ref[...] = b
    bcol_ref[...] = jnp.broadcast_to(b, (b.shape[0], 16))


def _mm2_body(pp_ref, a_ref, b_ref, w_ref, hp2_ref):
    pp = pp_ref[...]
    h = jnp.maximum((pp[0] + pp[1]) * b_ref[...], 0.0)
    mm = jnp.dot(h, w_ref[...], preferred_element_type=jnp.float32)
    hp2_ref[...] = mm * a_ref[...]


def _dec_body(gpb_ref, bgb_ref, gpa_ref, bga_ref, out_ref):
    gpb = gpb_ref[...]
    zsb = (gpb[0] + gpb[1]) * bgb_ref[...]
    gpa = gpa_ref[...]
    zsa = ((gpa[0] + gpa[1]) * bga_ref[...])[:S]
    gram = lax.dot_general(zsb, zsa, (((1,), (1,)), ((), ())),
                           preferred_element_type=jnp.float32)
    sqb = jnp.sum(zsb * zsb, axis=1)
    sqa = jnp.sum(zsa * zsa, axis=1)
    d2 = jnp.maximum(sqb[:, None] + sqa[None, :] - 2.0 * gram, 0.0)
    out_ref[0] = gram
    out_ref[1] = jnp.sqrt(d2 + 1e-12)


def kernel(x, edge_index, sampled_nodes, W1, W2):
    f32 = jnp.float32
    ei3 = edge_index.reshape(2, TB, BATCH)
    sampw = jnp.pad(sampled_nodes, (0, SP - S)).reshape(16, SPT)
    zeros1 = jnp.zeros((STRIPE,), f32)
    zeros32 = jnp.zeros((STRIPE, 32), f32)
    zeros16 = jnp.zeros((STRIPE, 16), f32)

    degs = pl.kernel(
        _deg_body,
        out_type=jax.ShapeDtypeStruct((2, NP), f32),
        mesh=_MESH,
        compiler_params=_SC_PARAMS,
        scratch_types=[
            pltpu.VMEM((DBPT, BATCH), jnp.int32),
            pltpu.VMEM((BATCH,), f32),
            pltpu.VMEM_SHARED((NP,), f32),
            pltpu.SemaphoreType.DMA,
        ],
    )(ei3, zeros1)

    RB = 2048
    grid = NP // RB
    mmraw = pl.pallas_call(
        _mm1a_body,
        grid=(grid,),
        in_specs=[
            pl.BlockSpec((RB, 128), lambda i: (i, 0)),
            pl.BlockSpec((128, 32), lambda i: (0, 0)),
        ],
        out_specs=pl.BlockSpec((RB, 32), lambda i: (i, 0)),
        out_shape=jax.ShapeDtypeStruct((NP, 32), f32),
    )(x, W1)

    RB2 = 5120
    grid2 = NP // RB2
    hp1, a, b, bcol = pl.pallas_call(
        _mm1b_body,
        grid=(grid2,),
        in_specs=[
            pl.BlockSpec((RB2, 32), lambda i: (i, 0)),
            pl.BlockSpec((2, RB2), lambda i: (0, i)),
        ],
        out_specs=[
            pl.BlockSpec((RB2, 32), lambda i: (i, 0)),
            pl.BlockSpec((RB2, 1), lambda i: (i, 0)),
            pl.BlockSpec((RB2, 1), lambda i: (i, 0)),
            pl.BlockSpec((RB2, 16), lambda i: (i, 0)),
        ],
        out_shape=[
            jax.ShapeDtypeStruct((NP, 32), f32),
            jax.ShapeDtypeStruct((NP, 1), f32),
            jax.ShapeDtypeStruct((NP, 1), f32),
            jax.ShapeDtypeStruct((NP, 16), f32),
        ],
    )(mmraw, degs)

    p32 = pl.kernel(
        _prop_body,
        out_type=jax.ShapeDtypeStruct((2, NP, 32), f32),
        mesh=_MESH,
        compiler_params=_SC_PARAMS,
        scratch_types=(
            [pltpu.VMEM((CAP, BATCH), jnp.int32)] * 2
            + [pltpu.VMEM((BATCH, 32), f32)] * 8
            + [pltpu.VMEM_SHARED((NP, 32), f32)]
            + [pltpu.SemaphoreType.DMA] * 16
        ),
    )(hp1, ei3, zeros32)

    hp2 = pl.pallas_call(
        _mm2_body,
        grid=(grid2,),
        in_specs=[
            pl.BlockSpec((2, RB2, 32), lambda i: (0, i, 0)),
            pl.BlockSpec((RB2, 1), lambda i: (i, 0)),
            pl.BlockSpec((RB2, 1), lambda i: (i, 0)),
            pl.BlockSpec((32, 16), lambda i: (0, 0)),
        ],
        out_specs=pl.BlockSpec((RB2, 16), lambda i: (i, 0)),
        out_shape=jax.ShapeDtypeStruct((NP, 16), f32),
    )(p32, a, b, W2)

    gpart, bg = pl.kernel(
        _prop_gather_body,
        out_type=(jax.ShapeDtypeStruct((2, SP, 16), f32),
                  jax.ShapeDtypeStruct((SP, 16), f32)),
        mesh=_MESH,
        compiler_params=_SC_PARAMS,
        scratch_types=(
            [pltpu.VMEM((CAP, BATCH), jnp.int32)] * 2
            + [pltpu.VMEM((BATCH, 16), f32)] * 8
            + [pltpu.VMEM_SHARED((NP, 16), f32)]
            + [pltpu.VMEM((SPT,), jnp.int32),
               pltpu.VMEM((SPT, 16), f32),
               pltpu.VMEM((SPT, 16), f32)]
            + [pltpu.SemaphoreType.DMA] * 16
        ),
    )(hp2, ei3, zeros16, bcol, sampw)

    RBD = 200
    out = pl.pallas_call(
        _dec_body,
        grid=(S // RBD,),
        in_specs=[
            pl.BlockSpec((2, RBD, 16), lambda i: (0, i, 0)),
            pl.BlockSpec((RBD, 16), lambda i: (i, 0)),
            pl.BlockSpec((2, SP, 16), lambda i: (0, 0, 0)),
            pl.BlockSpec((SP, 16), lambda i: (0, 0)),
        ],
        out_specs=pl.BlockSpec((2, RBD, S), lambda i: (0, i, 0)),
        out_shape=jax.ShapeDtypeStruct((2, S, S), f32),
    )(gpart, bg, gpart, bg)

    return out.reshape(2, S * S)


# speedup vs baseline: 1.0689x; 1.0689x over previous
import jax
import jax.numpy as jnp
from jax import lax
from jax.experimental import pallas as pl
from jax.experimental.pallas import tpu as pltpu
from jax.experimental.pallas import tpu_sc as plsc

N = 10000
NP = 10240
E = 320000
BATCH = 128
TB = E // BATCH
CAP = 80
DBPT = 157
STRIPE = NP // 16
S = 1000
SP = 1024
SPT = SP // 16

_MESH = plsc.VectorSubcoreMesh(
    core_axis_name="c", subcore_axis_name="s", num_cores=2, num_subcores=16)
_SC_PARAMS = pltpu.CompilerParams(use_tc_tiling_on_sc=False)


def _deg_body(ei3, zeros1, out, idx_v, ones_v, acc, sem):
    cid = lax.axis_index("c")
    sid = lax.axis_index("s")
    for i in range(8):
        ones_v[pl.ds(i * 16, 16)] = jnp.full((16,), 1.0, jnp.float32)
    pltpu.sync_copy(zeros1, acc.at[pl.ds(sid * STRIPE, STRIPE)])
    plane = jnp.where(cid == 0, 1, 0)
    base = 156 * sid + jnp.minimum(sid, 4)
    nb = jnp.where(sid < 4, 157, 156)

    @pl.when(sid < 4)
    def _():
        pltpu.sync_copy(ei3.at[plane, pl.ds(base, 157)], idx_v)

    @pl.when(sid >= 4)
    def _():
        pltpu.sync_copy(ei3.at[plane, pl.ds(base, 156)], idx_v.at[pl.ds(0, 156)])

    plsc.subcore_barrier()

    def body(j, carry):
        @pl.when(j < nb)
        def _():
            pltpu.async_copy(ones_v, acc.at[idx_v.at[j]], sem, add=True)
        return carry

    lax.fori_loop(0, DBPT, body, 0)

    def drain(j, carry):
        @pl.when(j < nb)
        def _():
            pltpu.make_async_copy(ones_v, acc.at[idx_v.at[j]], sem).wait()
        return carry

    lax.fori_loop(0, DBPT, drain, 0)
    plsc.subcore_barrier()
    pltpu.sync_copy(acc.at[pl.ds(sid * STRIPE, STRIPE)],
                    out.at[cid, pl.ds(sid * STRIPE, STRIPE)])


def _sweep_prologue(wid, ei3, idxs, idxd):
    base = 80 * jnp.minimum(wid, 17) + 76 * jnp.maximum(wid - 17, 0)
    nb = jnp.where(wid < 17, CAP, CAP - 4)

    @pl.when(wid < 17)
    def _():
        pltpu.sync_copy(ei3.at[0, pl.ds(base, CAP)], idxs)
        pltpu.sync_copy(ei3.at[1, pl.ds(base, CAP)], idxd)

    @pl.when(wid >= 17)
    def _():
        pltpu.sync_copy(ei3.at[0, pl.ds(base, CAP - 4)], idxs.at[pl.ds(0, CAP - 4)])
        pltpu.sync_copy(ei3.at[1, pl.ds(base, CAP - 4)], idxd.at[pl.ds(0, CAP - 4)])

    return nb


def _edge_sweep(feat, idxs, idxd, rows, acc, sems, nb):
    nd = len(rows)
    for b in range(nd):
        pltpu.async_copy(feat.at[idxs.at[b]], rows[b], sems[b])

    def body(qq, carry):
        j0 = nd * qq
        for b in range(nd):
            j = j0 + b

            @pl.when(j < nb)
            def _(b=b, j=j):
                pltpu.make_async_copy(feat.at[idxs.at[j]], rows[b], sems[b]).wait()
                pltpu.sync_copy(rows[b], acc.at[idxd.at[j]], add=True)

                @pl.when(j + nd < nb)
                def _():
                    pltpu.async_copy(feat.at[idxs.at[j + nd]], rows[b], sems[b])

        return carry

    lax.fori_loop(0, (CAP + nd - 1) // nd, body, 0)


def _prop_body(feat, ei3, zerosf, out, *scr):
    idxs, idxd = scr[0], scr[1]
    rows, acc = list(scr[2:10]), scr[10]
    sems = list(scr[11:19])
    cid = lax.axis_index("c")
    sid = lax.axis_index("s")
    wid = cid * 16 + sid
    pltpu.sync_copy(zerosf, acc.at[pl.ds(sid * STRIPE, STRIPE)])
    nb = _sweep_prologue(wid, ei3, idxs, idxd)
    plsc.subcore_barrier()
    _edge_sweep(feat, idxs, idxd, rows, acc, sems, nb)
    plsc.subcore_barrier()
    pltpu.sync_copy(acc.at[pl.ds(sid * STRIPE, STRIPE)],
                    out.at[cid, pl.ds(sid * STRIPE, STRIPE)])


def _prop_gather_body(feat, ei3, zerosf, bcol, sampw, gpart, bg, *scr):
    idxs, idxd = scr[0], scr[1]
    rows, acc = list(scr[2:10]), scr[10]
    sidx, srows, brows = scr[11], scr[12], scr[13]
    sems = list(scr[14:22])
    cid = lax.axis_index("c")
    sid = lax.axis_index("s")
    wid = cid * 16 + sid
    pltpu.sync_copy(zerosf, acc.at[pl.ds(sid * STRIPE, STRIPE)])
    nb = _sweep_prologue(wid, ei3, idxs, idxd)
    pltpu.sync_copy(sampw.at[sid], sidx)
    plsc.subcore_barrier()
    _edge_sweep(feat, idxs, idxd, rows, acc, sems, nb)
    plsc.subcore_barrier()
    pltpu.async_copy(acc.at[sidx], srows, sems[0]).wait()
    pltpu.sync_copy(srows, gpart.at[cid, pl.ds(sid * SPT, SPT)])

    @pl.when(cid == 0)
    def _():
        pltpu.async_copy(bcol.at[sidx], brows, sems[1]).wait()
        pltpu.sync_copy(brows, bg.at[pl.ds(sid * SPT, SPT)])


def _mm1a_body(x_ref, w_ref, mm_ref):
    mm_ref[...] = jnp.dot(x_ref[...], w_ref[...],
                          preferred_element_type=jnp.float32)


def _mm1b_body(mm_ref, deg_ref, hp1_ref, a_ref, b_ref, bcol_ref):
    deg = deg_ref[...]
    a = lax.rsqrt(jnp.maximum(deg[1], 1.0)).reshape(-1, 1)
    b = lax.rsqrt(jnp.maximum(deg[0], 1.0)).reshape(-1, 1)
    hp1_ref[...] = mm_ref[...] * a
    a_ref[...] = a
    b_ref[...] = b
    bcol_ref[...] = jnp.broadcast_to(b, (b.shape[0], 16))


def _mm2_body(pp_ref, a_ref, b_ref, w_ref, hp2_ref):
    pp = pp_ref[...]
    h = jnp.maximum((pp[0] + pp[1]) * b_ref[...], 0.0)
    mm = jnp.dot(h, w_ref[...], preferred_element_type=jnp.float32)
    hp2_ref[...] = mm * a_ref[...]


def _dec_body(gpb_ref, bgb_ref, gpa_ref, bga_ref, out_ref):
    gpb = gpb_ref[...]
    zsb = (gpb[0] + gpb[1]) * bgb_ref[...]
    gpa = gpa_ref[...]
    zsa = ((gpa[0] + gpa[1]) * bga_ref[...])[:S]
    gram = lax.dot_general(zsb, zsa, (((1,), (1,)), ((), ())),
                           preferred_element_type=jnp.float32)
    sqb = jnp.sum(zsb * zsb, axis=1)
    sqa = jnp.sum(zsa * zsa, axis=1)
    d2 = jnp.maximum(sqb[:, None] + sqa[None, :] - 2.0 * gram, 0.0)
    out_ref[0] = gram
    out_ref[1] = jnp.sqrt(d2 + 1e-12)


def kernel(x, edge_index, sampled_nodes, W1, W2):
    f32 = jnp.float32
    ei3 = edge_index.reshape(2, TB, BATCH)
    sampw = jnp.pad(sampled_nodes, (0, SP - S)).reshape(16, SPT)
    zeros1 = jnp.zeros((STRIPE,), f32)
    zeros32 = jnp.zeros((STRIPE, 32), f32)
    zeros16 = jnp.zeros((STRIPE, 16), f32)

    degs = pl.kernel(
        _deg_body,
        out_type=jax.ShapeDtypeStruct((2, NP), f32),
        mesh=_MESH,
        compiler_params=_SC_PARAMS,
        scratch_types=[
            pltpu.VMEM((DBPT, BATCH), jnp.int32),
            pltpu.VMEM((BATCH,), f32),
            pltpu.VMEM_SHARED((NP,), f32),
            pltpu.SemaphoreType.DMA,
        ],
    )(ei3, zeros1)

    RB = 2048
    grid = NP // RB
    mmraw = pl.pallas_call(
        _mm1a_body,
        grid=(grid,),
        in_specs=[
            pl.BlockSpec((RB, 128), lambda i: (i, 0)),
            pl.BlockSpec((128, 32), lambda i: (0, 0)),
        ],
        out_specs=pl.BlockSpec((RB, 32), lambda i: (i, 0)),
        out_shape=jax.ShapeDtypeStruct((NP, 32), f32),
    )(x, W1)

    RB2 = 5120
    grid2 = NP // RB2
    hp1, a, b, bcol = pl.pallas_call(
        _mm1b_body,
        grid=(grid2,),
        in_specs=[
            pl.BlockSpec((RB2, 32), lambda i: (i, 0)),
            pl.BlockSpec((2, RB2), lambda i: (0, i)),
        ],
        out_specs=[
            pl.BlockSpec((RB2, 32), lambda i: (i, 0)),
            pl.BlockSpec((RB2, 1), lambda i: (i, 0)),
            pl.BlockSpec((RB2, 1), lambda i: (i, 0)),
            pl.BlockSpec((RB2, 16), lambda i: (i, 0)),
        ],
        out_shape=[
            jax.ShapeDtypeStruct((NP, 32), f32),
            jax.ShapeDtypeStruct((NP, 1), f32),
            jax.ShapeDtypeStruct((NP, 1), f32),
            jax.ShapeDtypeStruct((NP, 16), f32),
        ],
    )(mmraw, degs)

    p32 = pl.kernel(
        _prop_body,
        out_type=jax.ShapeDtypeStruct((2, NP, 32), f32),
        mesh=_MESH,
        compiler_params=_SC_PARAMS,
        scratch_types=(
            [pltpu.VMEM((CAP, BATCH), jnp.int32)] * 2
            + [pltpu.VMEM((BATCH, 32), f32)] * 8
            + [pltpu.VMEM_SHARED((NP, 32), f32)]
            + [pltpu.SemaphoreType.DMA] * 8
        ),
    )(hp1, ei3, zeros32)

    hp2 = pl.pallas_call(
        _mm2_body,
        grid=(grid2,),
        in_specs=[
            pl.BlockSpec((2, RB2, 32), lambda i: (0, i, 0)),
            pl.BlockSpec((RB2, 1), lambda i: (i, 0)),
            pl.BlockSpec((RB2, 1), lambda i: (i, 0)),
            pl.BlockSpec((32, 16), lambda i: (0, 0)),
        ],
        out_specs=pl.BlockSpec((RB2, 16), lambda i: (i, 0)),
        out_shape=jax.ShapeDtypeStruct((NP, 16), f32),
    )(p32, a, b, W2)

    gpart, bg = pl.kernel(
        _prop_gather_body,
        out_type=(jax.ShapeDtypeStruct((2, SP, 16), f32),
                  jax.ShapeDtypeStruct((SP, 16), f32)),
        mesh=_MESH,
        compiler_params=_SC_PARAMS,
        scratch_types=(
            [pltpu.VMEM((CAP, BATCH), jnp.int32)] * 2
            + [pltpu.VMEM((BATCH, 16), f32)] * 8
            + [pltpu.VMEM_SHARED((NP, 16), f32)]
            + [pltpu.VMEM((SPT,), jnp.int32),
               pltpu.VMEM((SPT, 16), f32),
               pltpu.VMEM((SPT, 16), f32)]
            + [pltpu.SemaphoreType.DMA] * 8
        ),
    )(hp2, ei3, zeros16, bcol, sampw)

    RBD = 200
    out = pl.pallas_call(
        _dec_body,
        grid=(S // RBD,),
        in_specs=[
            pl.BlockSpec((2, RBD, 16), lambda i: (0, i, 0)),
            pl.BlockSpec((RBD, 16), lambda i: (i, 0)),
            pl.BlockSpec((2, SP, 16), lambda i: (0, 0, 0)),
            pl.BlockSpec((SP, 16), lambda i: (0, 0)),
        ],
        out_specs=pl.BlockSpec((2, RBD, S), lambda i: (0, i, 0)),
        out_shape=jax.ShapeDtypeStruct((2, S, S), f32),
    )(gpart, bg, gpart, bg)

    return out.reshape(2, S * S)
